# baseline (speedup 1.0000x reference)
import functools

import jax
import jax.numpy as jnp
import numpy as np
from jax.experimental import pallas as pl
from jax.experimental.pallas import tpu as pltpu

N_HEADS = 16
HEAD_DIM = 64
IDX_HEADS = 4
IDX_DIM = 64
TOP_K = 256

RB = 256
KC = 256
QB = 256

_SIGN = np.int32(-2**31)
_NEG = np.float32(-1e30)


def _dot(a, b):
    return jax.lax.dot_general(a, b, (((1,), (0,)), ((), ())),
                               preferred_element_type=jnp.float32)


def _dot_bf(a, b):
    return jax.lax.dot_general(a.astype(jnp.bfloat16),
                               b.astype(jnp.bfloat16),
                               (((1,), (0,)), ((), ())),
                               preferred_element_type=jnp.float32)


def _rope_apply(q, cos_t, sin_t, even):
    z = jnp.zeros((q.shape[0], 1), q.dtype)
    r_left = jnp.concatenate([q[:, 1:], z], axis=1)
    r_right = jnp.concatenate([z, q[:, :-1]], axis=1)
    rot = jnp.where(even, -r_left, r_right)
    return q * cos_t + rot * sin_t


def _proj_kernel(x_ref, wqkv_ref, wiq_ref, wik_ref, wiw_ref, cos_ref, sin_ref,
                 q_ref, kt_ref, v_ref, iq_ref, ikt_ref, iw_ref):
    d = x_ref.shape[1]
    xb = x_ref[...]
    qkv = _dot(xb, wqkv_ref[...])
    q = qkv[:, :d]
    k = qkv[:, d:2 * d]
    v = qkv[:, 2 * d:]
    cos_t = jnp.concatenate([cos_ref[...]] * N_HEADS, axis=1)
    sin_t = jnp.concatenate([sin_ref[...]] * N_HEADS, axis=1)
    even = jax.lax.broadcasted_iota(jnp.int32, (1, d), 1) % 2 == 0
    q_ref[...] = _rope_apply(q, cos_t, sin_t, even)
    kt_ref[...] = _rope_apply(k, cos_t, sin_t, even).T
    v_ref[...] = v
    iq_ref[...] = _dot_bf(xb, wiq_ref[...])
    ikt_ref[...] = _dot_bf(xb, wik_ref[...]).T
    iw_ref[...] = _dot_bf(xb, wiw_ref[...])


def _attn_kernel(q_ref, iq_ref, iw_ref, kt_ref, v_ref, ikt_ref, wo_ref,
                 out_ref, sc_ref, at_ref, acc_ref):
    qb, s = q_ref.shape[0], kt_ref.shape[1]
    nkc = s // KC
    blk = pl.program_id(0)
    rows = blk * qb + jax.lax.broadcasted_iota(jnp.int32, (qb, 1), 0)
    cols = jax.lax.broadcasted_iota(jnp.int32, (qb, s), 1)
    causal = cols <= rows

    iq_b = [iq_ref[:, h * IDX_DIM:(h + 1) * IDX_DIM].astype(jnp.bfloat16)
            for h in range(IDX_HEADS)]
    iw_b = [iw_ref[:, h][:, None].astype(jnp.bfloat16).astype(jnp.float32)
            for h in range(IDX_HEADS)]
    for j in range(nkc):
        def _sc_chunk(j=j):
            ik_c = ikt_ref[:, j * KC:(j + 1) * KC].astype(jnp.bfloat16)
            sc = jnp.zeros((qb, KC), jnp.float32)
            for h in range(IDX_HEADS):
                logit = jax.lax.dot_general(
                    iq_b[h], ik_c, (((1,), (0,)), ((), ())),
                    preferred_element_type=jnp.float32)
                r_h = jnp.maximum(logit, 0.0).astype(jnp.bfloat16).astype(
                    jnp.float32)
                sc = sc + iw_b[h] * r_h
            sc_ref[:, j * KC:(j + 1) * KC] = sc
        if j == 0:
            _sc_chunk()
        else:
            pl.when(j * KC <= blk * qb + qb - 1)(_sc_chunk)
    scores = jnp.where(causal, sc_ref[...], -jnp.inf)

    kb = jax.lax.bitcast_convert_type(scores, jnp.int32)
    keys = jnp.where(kb >= 0, kb, jnp.bitwise_xor(jnp.bitwise_not(kb), _SIGN))
    k_eff = jnp.minimum(np.float32(TOP_K), (rows + 1).astype(jnp.float32))
    cand = jnp.zeros((qb, 1), jnp.int32)
    for bit in range(31, -1, -1):
        bitc = np.int32(-2**31) if bit == 31 else np.int32(1 << bit)
        t_u = cand | bitc
        t_i = t_u ^ _SIGN
        cnt = jnp.sum(jnp.where(keys >= t_i, 1.0, 0.0), axis=1, keepdims=True)
        cand = jnp.where(cnt >= k_eff, t_u, cand)
    v_thresh = cand ^ _SIGN

    gt = keys > v_thresh
    c_gt = jnp.sum(jnp.where(gt, 1.0, 0.0), axis=1, keepdims=True)
    r = k_eff - c_gt
    eq = keys == v_thresh
    rank = jnp.where(eq, 1.0, 0.0)
    sh = 1
    while sh < s:
        z = jnp.zeros((qb, sh), jnp.float32)
        rank = rank + jnp.concatenate([z, rank[:, :-sh]], axis=1)
        sh *= 2
    sel = gt | (eq & (rank <= r))
    bias = jnp.where(sel, 0.0, _NEG)

    outs = []
    scale = np.float32(1.0 / np.sqrt(HEAD_DIM))
    last = blk * qb + qb - 1
    for h in range(N_HEADS):
        qh = q_ref[:, h * HEAD_DIM:(h + 1) * HEAD_DIM].astype(jnp.bfloat16)
        for j in range(nkc):
            def _qk_chunk(j=j):
                kh_c = kt_ref[h * HEAD_DIM:(h + 1) * HEAD_DIM,
                              j * KC:(j + 1) * KC].astype(jnp.bfloat16)
                at_ref[:, j * KC:(j + 1) * KC] = jax.lax.dot_general(
                    qh, kh_c, (((1,), (0,)), ((), ())),
                    preferred_element_type=jnp.float32)
            if j == 0:
                _qk_chunk()
            else:
                pl.when(j * KC <= last)(_qk_chunk)
        att = jnp.where(sel, at_ref[...] * scale, _NEG)
        m = jnp.max(att, axis=1, keepdims=True)
        p = jnp.exp(att - m)
        l = jnp.sum(p, axis=1, keepdims=True)
        p_b = p.astype(jnp.bfloat16)
        acc_ref[...] = jnp.zeros((qb, HEAD_DIM), jnp.float32)
        for j in range(nkc):
            def _pv_chunk(j=j):
                v_c = v_ref[j * KC:(j + 1) * KC,
                            h * HEAD_DIM:(h + 1) * HEAD_DIM].astype(
                                jnp.bfloat16)
                acc_ref[...] += jax.lax.dot_general(
                    p_b[:, j * KC:(j + 1) * KC], v_c,
                    (((1,), (0,)), ((), ())),
                    preferred_element_type=jnp.float32)
            if j == 0:
                _pv_chunk()
            else:
                pl.when(j * KC <= last)(_pv_chunk)
        outs.append(acc_ref[...] / l)
    ob = jnp.concatenate(outs, axis=1)
    out_ref[...] = _dot(ob, wo_ref[...])


@functools.partial(jax.jit, static_argnames=())
def kernel(x, W_qkv, W_o, W_iq, W_ik, W_iw):
    b, s, d = x.shape
    x2 = x.reshape(s, d)
    W_iw_pad = jnp.pad(W_iw, ((0, 0), (0, 128 - IDX_HEADS)))

    theta = 1.0 / (10000.0 ** (jnp.arange(0, HEAD_DIM, 2, dtype=jnp.float32)
                               / HEAD_DIM))
    pos = jnp.arange(s, dtype=jnp.float32)
    idx_theta = pos[:, None] * theta[None, :]
    cos2 = jnp.repeat(jnp.cos(idx_theta), 2, axis=1)
    sin2 = jnp.repeat(jnp.sin(idx_theta), 2, axis=1)

    nb = s // RB
    full = lambda shape: pl.BlockSpec(shape, lambda i: (0, 0))
    rowblk = lambda w: pl.BlockSpec((RB, w), lambda i: (i, 0))
    colblk = lambda hgt: pl.BlockSpec((hgt, RB), lambda i: (0, i))

    qr, krt, v, iq, ikt, iw = pl.pallas_call(
        _proj_kernel,
        grid=(nb,),
        in_specs=[rowblk(d), full((d, 3 * d)), full((d, IDX_HEADS * IDX_DIM)),
                  full((d, IDX_DIM)), full((d, 128)), rowblk(HEAD_DIM),
                  rowblk(HEAD_DIM)],
        out_specs=[rowblk(d), colblk(d), rowblk(d),
                   rowblk(IDX_HEADS * IDX_DIM), colblk(IDX_DIM), rowblk(128)],
        out_shape=[jax.ShapeDtypeStruct((s, d), jnp.float32),
                   jax.ShapeDtypeStruct((d, s), jnp.float32),
                   jax.ShapeDtypeStruct((s, d), jnp.float32),
                   jax.ShapeDtypeStruct((s, IDX_HEADS * IDX_DIM), jnp.float32),
                   jax.ShapeDtypeStruct((IDX_DIM, s), jnp.float32),
                   jax.ShapeDtypeStruct((s, 128), jnp.float32)],
    )(x2, W_qkv, W_iq, W_ik, W_iw_pad, cos2, sin2)

    nqb = s // QB
    out = pl.pallas_call(
        _attn_kernel,
        grid=(nqb,),
        in_specs=[pl.BlockSpec((QB, d), lambda i: (i, 0)),
                  pl.BlockSpec((QB, IDX_HEADS * IDX_DIM), lambda i: (i, 0)),
                  pl.BlockSpec((QB, 128), lambda i: (i, 0)),
                  full((d, s)), full((s, d)), full((IDX_DIM, s)),
                  full((d, d))],
        out_specs=pl.BlockSpec((QB, d), lambda i: (i, 0)),
        out_shape=jax.ShapeDtypeStruct((s, d), jnp.float32),
        scratch_shapes=[pltpu.VMEM((QB, s), jnp.float32),
                        pltpu.VMEM((QB, s), jnp.float32),
                        pltpu.VMEM((QB, HEAD_DIM), jnp.float32)],
    )(qr, iq, iw, krt, v, ikt, W_o)

    return out.reshape(b, s, d)


# speedup vs baseline: 2.0049x; 2.0049x over previous
import functools

import jax
import jax.numpy as jnp
import numpy as np
from jax.experimental import pallas as pl
from jax.experimental.pallas import tpu as pltpu

N_HEADS = 16
HEAD_DIM = 64
IDX_HEADS = 4
IDX_DIM = 64
TOP_K = 256

RB = 256
KC = 256
QB = 256

_SIGN = np.int32(-2**31)
_NEG = np.float32(-1e30)


def _dot(a, b):
    return jax.lax.dot_general(a, b, (((1,), (0,)), ((), ())),
                               preferred_element_type=jnp.float32)


def _dot_bf(a, b):
    return jax.lax.dot_general(a.astype(jnp.bfloat16),
                               b.astype(jnp.bfloat16),
                               (((1,), (0,)), ((), ())),
                               preferred_element_type=jnp.float32)


def _rope_apply(q, cos_t, sin_t, even):
    z = jnp.zeros((q.shape[0], 1), q.dtype)
    r_left = jnp.concatenate([q[:, 1:], z], axis=1)
    r_right = jnp.concatenate([z, q[:, :-1]], axis=1)
    rot = jnp.where(even, -r_left, r_right)
    return q * cos_t + rot * sin_t


def _proj_kernel(x_ref, wqkv_ref, wiq_ref, wik_ref, wiw_ref, cos_ref, sin_ref,
                 q_ref, kt_ref, v_ref, iq_ref, ikt_ref, iw_ref):
    d = x_ref.shape[1]
    xb = x_ref[...]
    qkv = _dot(xb, wqkv_ref[...])
    q = qkv[:, :d]
    k = qkv[:, d:2 * d]
    v = qkv[:, 2 * d:]
    cos_t = jnp.concatenate([cos_ref[...]] * N_HEADS, axis=1)
    sin_t = jnp.concatenate([sin_ref[...]] * N_HEADS, axis=1)
    even = jax.lax.broadcasted_iota(jnp.int32, (1, d), 1) % 2 == 0
    q_ref[...] = _rope_apply(q, cos_t, sin_t, even)
    kt_ref[...] = _rope_apply(k, cos_t, sin_t, even).T
    v_ref[...] = v
    iq_ref[...] = _dot_bf(xb, wiq_ref[...])
    ikt_ref[...] = _dot_bf(xb, wik_ref[...]).T
    iw_ref[...] = _dot_bf(xb, wiw_ref[...])


def _attn_kernel(q_ref, iq_ref, iw_ref, kt_ref, v_ref, ikt_ref, wo_ref,
                 out_ref):
    qb, s = q_ref.shape[0], kt_ref.shape[1]
    blk = pl.program_id(0)
    rows = blk * qb + jax.lax.broadcasted_iota(jnp.int32, (qb, 1), 0)
    cols = jax.lax.broadcasted_iota(jnp.int32, (qb, s), 1)
    causal = cols <= rows

    scores = jnp.zeros((qb, s), jnp.float32)
    ikt_b = ikt_ref[...].astype(jnp.bfloat16)
    for h in range(IDX_HEADS):
        iq_h = iq_ref[:, h * IDX_DIM:(h + 1) * IDX_DIM]
        logit = jax.lax.dot_general(iq_h.astype(jnp.bfloat16), ikt_b,
                                    (((1,), (0,)), ((), ())),
                                    preferred_element_type=jnp.float32)
        w_h = iw_ref[:, h][:, None].astype(jnp.bfloat16).astype(jnp.float32)
        r_h = jnp.maximum(logit, 0.0).astype(jnp.bfloat16).astype(jnp.float32)
        scores = scores + w_h * r_h
    scores = jnp.where(causal, scores, -jnp.inf)

    kb = jax.lax.bitcast_convert_type(scores, jnp.int32)
    keys = jnp.where(kb >= 0, kb, jnp.bitwise_xor(jnp.bitwise_not(kb), _SIGN))
    k_eff = jnp.minimum(np.float32(TOP_K), (rows + 1).astype(jnp.float32))
    cand = jnp.zeros((qb, 1), jnp.int32)
    for bit in range(31, -1, -1):
        bitc = np.int32(-2**31) if bit == 31 else np.int32(1 << bit)
        t_u = cand | bitc
        t_i = t_u ^ _SIGN
        cnt = jnp.sum(jnp.where(keys >= t_i, 1.0, 0.0), axis=1, keepdims=True)
        cand = jnp.where(cnt >= k_eff, t_u, cand)
    v_thresh = cand ^ _SIGN

    gt = keys > v_thresh
    c_gt = jnp.sum(jnp.where(gt, 1.0, 0.0), axis=1, keepdims=True)
    r = k_eff - c_gt
    eq = keys == v_thresh
    rank = jnp.where(eq, 1.0, 0.0)
    sh = 1
    while sh < s:
        z = jnp.zeros((qb, sh), jnp.float32)
        rank = rank + jnp.concatenate([z, rank[:, :-sh]], axis=1)
        sh *= 2
    sel = gt | (eq & (rank <= r))
    bias = jnp.where(sel, 0.0, _NEG)

    outs = []
    scale = np.float32(1.0 / np.sqrt(HEAD_DIM))
    for h in range(N_HEADS):
        qh = q_ref[:, h * HEAD_DIM:(h + 1) * HEAD_DIM]
        kh_t = kt_ref[h * HEAD_DIM:(h + 1) * HEAD_DIM, :]
        att = _dot_bf(qh, kh_t) * scale + bias
        m = jnp.max(att, axis=1, keepdims=True)
        p = jnp.exp(att - m)
        l = jnp.sum(p, axis=1, keepdims=True)
        outs.append(_dot_bf(p, v_ref[:, h * HEAD_DIM:(h + 1) * HEAD_DIM]) / l)
    ob = jnp.concatenate(outs, axis=1)
    out_ref[...] = _dot(ob, wo_ref[...])


@functools.partial(jax.jit, static_argnames=())
def kernel(x, W_qkv, W_o, W_iq, W_ik, W_iw):
    b, s, d = x.shape
    x2 = x.reshape(s, d)
    W_iw_pad = jnp.pad(W_iw, ((0, 0), (0, 128 - IDX_HEADS)))

    theta = 1.0 / (10000.0 ** (jnp.arange(0, HEAD_DIM, 2, dtype=jnp.float32)
                               / HEAD_DIM))
    pos = jnp.arange(s, dtype=jnp.float32)
    idx_theta = pos[:, None] * theta[None, :]
    cos2 = jnp.repeat(jnp.cos(idx_theta), 2, axis=1)
    sin2 = jnp.repeat(jnp.sin(idx_theta), 2, axis=1)

    nb = s // RB
    full = lambda shape: pl.BlockSpec(shape, lambda i: (0, 0))
    rowblk = lambda w: pl.BlockSpec((RB, w), lambda i: (i, 0))
    colblk = lambda hgt: pl.BlockSpec((hgt, RB), lambda i: (0, i))

    qr, krt, v, iq, ikt, iw = pl.pallas_call(
        _proj_kernel,
        grid=(nb,),
        in_specs=[rowblk(d), full((d, 3 * d)), full((d, IDX_HEADS * IDX_DIM)),
                  full((d, IDX_DIM)), full((d, 128)), rowblk(HEAD_DIM),
                  rowblk(HEAD_DIM)],
        out_specs=[rowblk(d), colblk(d), rowblk(d),
                   rowblk(IDX_HEADS * IDX_DIM), colblk(IDX_DIM), rowblk(128)],
        out_shape=[jax.ShapeDtypeStruct((s, d), jnp.float32),
                   jax.ShapeDtypeStruct((d, s), jnp.float32),
                   jax.ShapeDtypeStruct((s, d), jnp.float32),
                   jax.ShapeDtypeStruct((s, IDX_HEADS * IDX_DIM), jnp.float32),
                   jax.ShapeDtypeStruct((IDX_DIM, s), jnp.float32),
                   jax.ShapeDtypeStruct((s, 128), jnp.float32)],
    )(x2, W_qkv, W_iq, W_ik, W_iw_pad, cos2, sin2)

    nqb = s // QB
    out = pl.pallas_call(
        _attn_kernel,
        grid=(nqb,),
        in_specs=[pl.BlockSpec((QB, d), lambda i: (i, 0)),
                  pl.BlockSpec((QB, IDX_HEADS * IDX_DIM), lambda i: (i, 0)),
                  pl.BlockSpec((QB, 128), lambda i: (i, 0)),
                  full((d, s)), full((s, d)), full((IDX_DIM, s)),
                  full((d, d))],
        out_specs=pl.BlockSpec((QB, d), lambda i: (i, 0)),
        out_shape=jax.ShapeDtypeStruct((s, d), jnp.float32),
    )(qr, iq, iw, krt, v, ikt, W_o)

    return out.reshape(b, s, d)


# speedup vs baseline: 2.1176x; 1.0562x over previous
import functools

import jax
import jax.numpy as jnp
import numpy as np
from jax.experimental import pallas as pl
from jax.experimental.pallas import tpu as pltpu

N_HEADS = 16
HEAD_DIM = 64
IDX_HEADS = 4
IDX_DIM = 64
TOP_K = 256

RB = 256
KC = 256
QB = 256

_SIGN = np.int32(-2**31)
_NEG = np.float32(-1e30)


def _dot(a, b):
    return jax.lax.dot_general(a, b, (((1,), (0,)), ((), ())),
                               preferred_element_type=jnp.float32)


def _dot_bf(a, b):
    return jax.lax.dot_general(a.astype(jnp.bfloat16),
                               b.astype(jnp.bfloat16),
                               (((1,), (0,)), ((), ())),
                               preferred_element_type=jnp.float32)


def _rope_apply(q, cos_t, sin_t, even):
    z = jnp.zeros((q.shape[0], 1), q.dtype)
    r_left = jnp.concatenate([q[:, 1:], z], axis=1)
    r_right = jnp.concatenate([z, q[:, :-1]], axis=1)
    rot = jnp.where(even, -r_left, r_right)
    return q * cos_t + rot * sin_t


def _proj_kernel(x_ref, wqkv_ref, wiq_ref, wik_ref, wiw_ref, cos_ref, sin_ref,
                 q_ref, kt_ref, v_ref, iq_ref, ikt_ref, iw_ref):
    d = x_ref.shape[1]
    xb = x_ref[...]
    qkv = _dot(xb, wqkv_ref[...])
    q = qkv[:, :d]
    k = qkv[:, d:2 * d]
    v = qkv[:, 2 * d:]
    cos_t = jnp.concatenate([cos_ref[...]] * N_HEADS, axis=1)
    sin_t = jnp.concatenate([sin_ref[...]] * N_HEADS, axis=1)
    even = jax.lax.broadcasted_iota(jnp.int32, (1, d), 1) % 2 == 0
    q_ref[...] = _rope_apply(q, cos_t, sin_t, even)
    kt_ref[...] = _rope_apply(k, cos_t, sin_t, even).T
    v_ref[...] = v
    iq_ref[...] = _dot_bf(xb, wiq_ref[...])
    ikt_ref[...] = _dot_bf(xb, wik_ref[...]).T
    iw_ref[...] = _dot_bf(xb, wiw_ref[...])


def _attn_kernel(q_ref, iq_ref, iw_ref, kt_ref, v_ref, ikt_ref, wo_ref,
                 out_ref):
    qb, s = q_ref.shape[0], kt_ref.shape[1]
    blk = pl.program_id(0)
    rows = blk * qb + jax.lax.broadcasted_iota(jnp.int32, (qb, 1), 0)
    cols = jax.lax.broadcasted_iota(jnp.int32, (qb, s), 1)
    causal = cols <= rows

    scores = jnp.zeros((qb, s), jnp.float32)
    ikt_b = ikt_ref[...].astype(jnp.bfloat16)
    for h in range(IDX_HEADS):
        iq_h = iq_ref[:, h * IDX_DIM:(h + 1) * IDX_DIM]
        logit = jax.lax.dot_general(iq_h.astype(jnp.bfloat16), ikt_b,
                                    (((1,), (0,)), ((), ())),
                                    preferred_element_type=jnp.float32)
        w_h = iw_ref[:, h][:, None].astype(jnp.bfloat16).astype(jnp.float32)
        r_h = jnp.maximum(logit, 0.0).astype(jnp.bfloat16).astype(jnp.float32)
        scores = scores + w_h * r_h
    scores = jnp.where(causal, scores, -jnp.inf)

    kb = jax.lax.bitcast_convert_type(scores, jnp.int32)
    keys = jnp.where(kb >= 0, kb, jnp.bitwise_xor(jnp.bitwise_not(kb), _SIGN))
    k_eff = jnp.minimum(np.float32(TOP_K), (rows + 1).astype(jnp.float32))
    cand = jnp.zeros((qb, 1), jnp.int32)
    for bit in range(31, -1, -1):
        bitc = np.int32(-2**31) if bit == 31 else np.int32(1 << bit)
        t_u = cand | bitc
        t_i = t_u ^ _SIGN
        cnt = jnp.sum(jnp.where(keys >= t_i, 1.0, 0.0), axis=1, keepdims=True)
        cand = jnp.where(cnt >= k_eff, t_u, cand)
    v_thresh = cand ^ _SIGN

    gt = keys > v_thresh
    c_gt = jnp.sum(jnp.where(gt, 1.0, 0.0), axis=1, keepdims=True)
    r = k_eff - c_gt
    eq = keys == v_thresh
    rank = jnp.where(eq, 1.0, 0.0)
    sh = 1
    while sh < s:
        z = jnp.zeros((qb, sh), jnp.float32)
        rank = rank + jnp.concatenate([z, rank[:, :-sh]], axis=1)
        sh *= 2
    sel = gt | (eq & (rank <= r))
    bias = jnp.where(sel, 0.0, _NEG)

    outs = []
    scale = np.float32(1.0 / np.sqrt(HEAD_DIM))
    for h in range(N_HEADS):
        qh = q_ref[:, h * HEAD_DIM:(h + 1) * HEAD_DIM]
        kh_t = kt_ref[h * HEAD_DIM:(h + 1) * HEAD_DIM, :]
        att = _dot(qh, kh_t) * scale + bias
        m = jnp.max(att, axis=1, keepdims=True)
        p = jnp.exp(att - m)
        l = jnp.sum(p, axis=1, keepdims=True)
        outs.append(_dot(p, v_ref[:, h * HEAD_DIM:(h + 1) * HEAD_DIM]) / l)
    ob = jnp.concatenate(outs, axis=1)
    out_ref[...] = _dot(ob, wo_ref[...])


@functools.partial(jax.jit, static_argnames=())
def kernel(x, W_qkv, W_o, W_iq, W_ik, W_iw):
    b, s, d = x.shape
    x2 = x.reshape(s, d)
    W_iw_pad = jnp.pad(W_iw, ((0, 0), (0, 128 - IDX_HEADS)))

    theta = 1.0 / (10000.0 ** (jnp.arange(0, HEAD_DIM, 2, dtype=jnp.float32)
                               / HEAD_DIM))
    pos = jnp.arange(s, dtype=jnp.float32)
    idx_theta = pos[:, None] * theta[None, :]
    cos2 = jnp.repeat(jnp.cos(idx_theta), 2, axis=1)
    sin2 = jnp.repeat(jnp.sin(idx_theta), 2, axis=1)

    nb = s // RB
    full = lambda shape: pl.BlockSpec(shape, lambda i: (0, 0))
    rowblk = lambda w: pl.BlockSpec((RB, w), lambda i: (i, 0))
    colblk = lambda hgt: pl.BlockSpec((hgt, RB), lambda i: (0, i))

    qr, krt, v, iq, ikt, iw = pl.pallas_call(
        _proj_kernel,
        grid=(nb,),
        in_specs=[rowblk(d), full((d, 3 * d)), full((d, IDX_HEADS * IDX_DIM)),
                  full((d, IDX_DIM)), full((d, 128)), rowblk(HEAD_DIM),
                  rowblk(HEAD_DIM)],
        out_specs=[rowblk(d), colblk(d), rowblk(d),
                   rowblk(IDX_HEADS * IDX_DIM), colblk(IDX_DIM), rowblk(128)],
        out_shape=[jax.ShapeDtypeStruct((s, d), jnp.float32),
                   jax.ShapeDtypeStruct((d, s), jnp.float32),
                   jax.ShapeDtypeStruct((s, d), jnp.float32),
                   jax.ShapeDtypeStruct((s, IDX_HEADS * IDX_DIM), jnp.float32),
                   jax.ShapeDtypeStruct((IDX_DIM, s), jnp.float32),
                   jax.ShapeDtypeStruct((s, 128), jnp.float32)],
    )(x2, W_qkv, W_iq, W_ik, W_iw_pad, cos2, sin2)

    nqb = s // QB
    out = pl.pallas_call(
        _attn_kernel,
        grid=(nqb,),
        in_specs=[pl.BlockSpec((QB, d), lambda i: (i, 0)),
                  pl.BlockSpec((QB, IDX_HEADS * IDX_DIM), lambda i: (i, 0)),
                  pl.BlockSpec((QB, 128), lambda i: (i, 0)),
                  full((d, s)), full((s, d)), full((IDX_DIM, s)),
                  full((d, d))],
        out_specs=pl.BlockSpec((QB, d), lambda i: (i, 0)),
        out_shape=jax.ShapeDtypeStruct((s, d), jnp.float32),
    )(qr, iq, iw, krt, v, ikt, W_o)

    return out.reshape(b, s, d)


# speedup vs baseline: 2.3178x; 1.0946x over previous
import functools

import jax
import jax.numpy as jnp
import numpy as np
from jax.experimental import pallas as pl
from jax.experimental.pallas import tpu as pltpu

N_HEADS = 16
HEAD_DIM = 64
IDX_HEADS = 4
IDX_DIM = 64
TOP_K = 256

RB = 256
KC = 256
QB = 256

_SIGN = np.int32(-2**31)
_NEG = np.float32(-1e30)


def _dot(a, b):
    return jax.lax.dot_general(a, b, (((1,), (0,)), ((), ())),
                               preferred_element_type=jnp.float32)


def _dot_bf(a, b):
    return jax.lax.dot_general(a.astype(jnp.bfloat16),
                               b.astype(jnp.bfloat16),
                               (((1,), (0,)), ((), ())),
                               preferred_element_type=jnp.float32)


def _rope_apply(q, cos_t, sin_t, even):
    z = jnp.zeros((q.shape[0], 1), q.dtype)
    r_left = jnp.concatenate([q[:, 1:], z], axis=1)
    r_right = jnp.concatenate([z, q[:, :-1]], axis=1)
    rot = jnp.where(even, -r_left, r_right)
    return q * cos_t + rot * sin_t


def _proj_kernel(x_ref, wqkv_ref, wiq_ref, wik_ref, wiw_ref, cos_ref, sin_ref,
                 q_ref, kt_ref, v_ref, iq_ref, ikt_ref, iw_ref):
    d = x_ref.shape[1]
    xb = x_ref[...]
    qkv = _dot(xb, wqkv_ref[...])
    q = qkv[:, :d]
    k = qkv[:, d:2 * d]
    v = qkv[:, 2 * d:]
    cos_t = jnp.concatenate([cos_ref[...]] * N_HEADS, axis=1)
    sin_t = jnp.concatenate([sin_ref[...]] * N_HEADS, axis=1)
    even = jax.lax.broadcasted_iota(jnp.int32, (1, d), 1) % 2 == 0
    q_ref[...] = _rope_apply(q, cos_t, sin_t, even)
    kt_ref[...] = _rope_apply(k, cos_t, sin_t, even).T
    v_ref[...] = v
    iq_ref[...] = _dot_bf(xb, wiq_ref[...])
    ikt_ref[...] = _dot_bf(xb, wik_ref[...]).T
    iw_ref[...] = _dot_bf(xb, wiw_ref[...])


def _attn_kernel(q_ref, iq_ref, iw_ref, kt_ref, v_ref, ikt_ref, wo_ref,
                 out_ref, *, blk):
    qb, s = q_ref.shape[0], kt_ref.shape[1]
    rows = blk * qb + jax.lax.broadcasted_iota(jnp.int32, (qb, 1), 0)
    cols = jax.lax.broadcasted_iota(jnp.int32, (qb, s), 1)
    causal = cols <= rows

    scores = jnp.zeros((qb, s), jnp.float32)
    ikt_b = ikt_ref[...].astype(jnp.bfloat16)
    for h in range(IDX_HEADS):
        iq_h = iq_ref[:, h * IDX_DIM:(h + 1) * IDX_DIM]
        logit = jax.lax.dot_general(iq_h.astype(jnp.bfloat16), ikt_b,
                                    (((1,), (0,)), ((), ())),
                                    preferred_element_type=jnp.float32)
        w_h = iw_ref[:, h][:, None].astype(jnp.bfloat16).astype(jnp.float32)
        r_h = jnp.maximum(logit, 0.0).astype(jnp.bfloat16).astype(jnp.float32)
        scores = scores + w_h * r_h
    scores = jnp.where(causal, scores, -jnp.inf)

    kb = jax.lax.bitcast_convert_type(scores, jnp.int32)
    keys = jnp.where(kb >= 0, kb, jnp.bitwise_xor(jnp.bitwise_not(kb), _SIGN))
    k_eff = jnp.minimum(np.float32(TOP_K), (rows + 1).astype(jnp.float32))
    cand = jnp.zeros((qb, 1), jnp.int32)
    for bit in range(31, -1, -1):
        bitc = np.int32(-2**31) if bit == 31 else np.int32(1 << bit)
        t_u = cand | bitc
        t_i = t_u ^ _SIGN
        cnt = jnp.sum(jnp.where(keys >= t_i, 1.0, 0.0), axis=1, keepdims=True)
        cand = jnp.where(cnt >= k_eff, t_u, cand)
    v_thresh = cand ^ _SIGN

    gt = keys > v_thresh
    c_gt = jnp.sum(jnp.where(gt, 1.0, 0.0), axis=1, keepdims=True)
    r = k_eff - c_gt
    eq = keys == v_thresh
    rank = jnp.where(eq, 1.0, 0.0)
    sh = 1
    while sh < s:
        z = jnp.zeros((qb, sh), jnp.float32)
        rank = rank + jnp.concatenate([z, rank[:, :-sh]], axis=1)
        sh *= 2
    sel = gt | (eq & (rank <= r))
    bias = jnp.where(sel, 0.0, _NEG)

    outs = []
    scale = np.float32(1.0 / np.sqrt(HEAD_DIM))
    for h in range(N_HEADS):
        qh = q_ref[:, h * HEAD_DIM:(h + 1) * HEAD_DIM]
        kh_t = kt_ref[h * HEAD_DIM:(h + 1) * HEAD_DIM, :]
        att = _dot(qh, kh_t) * scale + bias
        m = jnp.max(att, axis=1, keepdims=True)
        p = jnp.exp(att - m)
        l = jnp.sum(p, axis=1, keepdims=True)
        outs.append(_dot(p, v_ref[:, h * HEAD_DIM:(h + 1) * HEAD_DIM]) / l)
    ob = jnp.concatenate(outs, axis=1)
    out_ref[...] = _dot(ob, wo_ref[...])


@functools.partial(jax.jit, static_argnames=())
def kernel(x, W_qkv, W_o, W_iq, W_ik, W_iw):
    b, s, d = x.shape
    x2 = x.reshape(s, d)
    W_iw_pad = jnp.pad(W_iw, ((0, 0), (0, 128 - IDX_HEADS)))

    theta = 1.0 / (10000.0 ** (jnp.arange(0, HEAD_DIM, 2, dtype=jnp.float32)
                               / HEAD_DIM))
    pos = jnp.arange(s, dtype=jnp.float32)
    idx_theta = pos[:, None] * theta[None, :]
    cos2 = jnp.repeat(jnp.cos(idx_theta), 2, axis=1)
    sin2 = jnp.repeat(jnp.sin(idx_theta), 2, axis=1)

    nb = s // RB
    full = lambda shape: pl.BlockSpec(shape, lambda i: (0, 0))
    rowblk = lambda w: pl.BlockSpec((RB, w), lambda i: (i, 0))
    colblk = lambda hgt: pl.BlockSpec((hgt, RB), lambda i: (0, i))

    qr, krt, v, iq, ikt, iw = pl.pallas_call(
        _proj_kernel,
        grid=(nb,),
        in_specs=[rowblk(d), full((d, 3 * d)), full((d, IDX_HEADS * IDX_DIM)),
                  full((d, IDX_DIM)), full((d, 128)), rowblk(HEAD_DIM),
                  rowblk(HEAD_DIM)],
        out_specs=[rowblk(d), colblk(d), rowblk(d),
                   rowblk(IDX_HEADS * IDX_DIM), colblk(IDX_DIM), rowblk(128)],
        out_shape=[jax.ShapeDtypeStruct((s, d), jnp.float32),
                   jax.ShapeDtypeStruct((d, s), jnp.float32),
                   jax.ShapeDtypeStruct((s, d), jnp.float32),
                   jax.ShapeDtypeStruct((s, IDX_HEADS * IDX_DIM), jnp.float32),
                   jax.ShapeDtypeStruct((IDX_DIM, s), jnp.float32),
                   jax.ShapeDtypeStruct((s, 128), jnp.float32)],
    )(x2, W_qkv, W_iq, W_ik, W_iw_pad, cos2, sin2)

    nqb = s // QB
    outs = []
    for i in range(nqb):
        ext = (i + 1) * QB
        qblk = lambda w, i=i: pl.BlockSpec((QB, w), lambda _, i=i: (i, 0))
        out_i = pl.pallas_call(
            functools.partial(_attn_kernel, blk=i),
            grid=(1,),
            in_specs=[qblk(d), qblk(IDX_HEADS * IDX_DIM), qblk(128),
                      full((d, ext)), full((ext, d)), full((IDX_DIM, ext)),
                      full((d, d))],
            out_specs=pl.BlockSpec((QB, d), lambda _: (0, 0)),
            out_shape=jax.ShapeDtypeStruct((QB, d), jnp.float32),
        )(qr, iq, iw, krt, v, ikt, W_o)
        outs.append(out_i)
    out = jnp.concatenate(outs, axis=0)

    return out.reshape(b, s, d)
